# Initial kernel scaffold; baseline (speedup 1.0000x reference)
import jax
import jax.numpy as jnp
from jax.experimental import pallas as pl


def kernel(X, emb, W, b):
    raise NotImplementedError("write your pallas kernel here")


# speedup vs baseline: 11.9321x; 11.9321x over previous
import functools

import jax
import jax.numpy as jnp
from jax import lax
from jax.experimental import pallas as pl
from jax.experimental.pallas import tpu as pltpu
from jax.experimental.pallas import tpu_sc as plsc

NUM_EMB = 1_000_000
EMBED_DIM = 32
BATCH = 16384
HIST = 200

NW = 32
ROWS_PER_W = BATCH // NW
CHUNK_ROWS = 128
N_CHUNKS = ROWS_PER_W // CHUNK_ROWS
IDX_PER_CHUNK = CHUNK_ROWS * HIST
IDX_ROWS = IDX_PER_CHUNK // 128
XROWS_PER_W = (BATCH * HIST) // 128 // NW


def _table_body(emb_ref, w_ref, b_ref, out_ref):
    out_ref[...] = (
        jnp.dot(emb_ref[...], w_ref[...], preferred_element_type=jnp.float32)
        + b_ref[0]
    )


def _make_table(emb, w_scaled, b_scaled):
    block_rows = 4000
    grid = (NUM_EMB // block_rows,)
    return pl.pallas_call(
        _table_body,
        grid=grid,
        in_specs=[
            pl.BlockSpec((block_rows, EMBED_DIM), lambda i: (i, 0)),
            pl.BlockSpec((EMBED_DIM, 1), lambda i: (0, 0)),
            pl.BlockSpec(memory_space=pltpu.SMEM),
        ],
        out_specs=pl.BlockSpec((block_rows, 1), lambda i: (i, 0)),
        out_shape=jax.ShapeDtypeStruct((NUM_EMB, 1), jnp.float32),
    )(emb, w_scaled, b_scaled)


def _gather_sum(x2, p):
    mesh = plsc.VectorSubcoreMesh(core_axis_name="c", subcore_axis_name="s")

    @functools.partial(
        pl.kernel,
        out_type=jax.ShapeDtypeStruct((BATCH,), jnp.float32),
        mesh=mesh,
        compiler_params=pltpu.CompilerParams(needs_layout_passes=False),
        scratch_types=[
            pltpu.VMEM((IDX_ROWS, 128), jnp.int32),
            pltpu.VMEM((IDX_PER_CHUNK,), jnp.float32),
            pltpu.VMEM((ROWS_PER_W,), jnp.float32),
            pltpu.SemaphoreType.DMA,
        ],
    )
    def body(x2_hbm, p_hbm, y_hbm, idx_v, vals_v, out_v, sem):
        wid = lax.axis_index("c") * 16 + lax.axis_index("s")
        iota200 = lax.iota(jnp.int32, 16) * HIST

        def chunk_body(c, carry):
            xrow = wid * XROWS_PER_W + c * IDX_ROWS
            pltpu.sync_copy(x2_hbm.at[pl.ds(xrow, IDX_ROWS)], idx_v)

            def fire(j, carry2):
                for k in range(8):
                    r = j * 8 + k
                    pltpu.async_copy(
                        p_hbm.at[idx_v.at[r]],
                        vals_v.at[pl.ds(pl.multiple_of(r * 128, 128), 128)],
                        sem,
                    )
                return carry2

            lax.fori_loop(0, IDX_ROWS // 8, fire, 0)
            pltpu.make_async_copy(
                p_hbm.at[pl.ds(0, IDX_PER_CHUNK)], vals_v, sem
            ).wait()

            for g in range(8):
                base = g * 16 * HIST

                def red(j, acc):
                    jb = base + j * 8
                    vs = [
                        plsc.load_gather(vals_v, [iota200 + (jb + k)])
                        for k in range(8)
                    ]
                    s = ((vs[0] + vs[1]) + (vs[2] + vs[3])) + (
                        (vs[4] + vs[5]) + (vs[6] + vs[7])
                    )
                    return acc + s

                acc = lax.fori_loop(
                    0, HIST // 8, red, jnp.zeros((16,), jnp.float32)
                )
                out_v[pl.ds(pl.multiple_of(c * CHUNK_ROWS + g * 16, 16), 16)] = acc
            return carry

        lax.fori_loop(0, N_CHUNKS, chunk_body, 0)
        pltpu.sync_copy(
            out_v, y_hbm.at[pl.ds(pl.multiple_of(wid * ROWS_PER_W, 512), ROWS_PER_W)]
        )

    return body(x2, p)


def kernel(X, emb, W, b):
    x2 = X.astype(jnp.int32).reshape(BATCH * HIST // 128, 128)
    w_scaled = (W.astype(jnp.float32) * (1.0 / HIST)).reshape(EMBED_DIM, 1)
    b_scaled = (b.astype(jnp.float32) * (1.0 / HIST)).reshape(1)
    p = _make_table(emb, w_scaled, b_scaled).reshape(NUM_EMB)
    y = _gather_sum(x2, p)
    return y.reshape(BATCH, 1)


# speedup vs baseline: 15.8905x; 1.3317x over previous
import functools

import jax
import jax.numpy as jnp
from jax import lax
from jax.experimental import pallas as pl
from jax.experimental.pallas import tpu as pltpu
from jax.experimental.pallas import tpu_sc as plsc

NUM_EMB = 1_000_000
EMBED_DIM = 32
BATCH = 16384
HIST = 200

NW = 32
ROWS_PER_W = BATCH // NW
CHUNK_ROWS = 128
N_CHUNKS = ROWS_PER_W // CHUNK_ROWS
IDX_PER_CHUNK = CHUNK_ROWS * HIST
IDX_ROWS = IDX_PER_CHUNK // 128
XROWS_PER_W = (BATCH * HIST) // 128 // NW

TBLOCK = 4000


def _table_body(emb_ref, w_ref, b_ref, out_ref):
    out_ref[0] = (
        lax.dot_general(
            w_ref[...],
            emb_ref[...],
            dimension_numbers=(((1,), (1,)), ((), ())),
            preferred_element_type=jnp.float32,
        )
        + b_ref[0]
    )


def _make_table(emb, w_scaled, b_scaled):
    grid = (NUM_EMB // TBLOCK,)
    return pl.pallas_call(
        _table_body,
        grid=grid,
        in_specs=[
            pl.BlockSpec((TBLOCK, EMBED_DIM), lambda i: (i, 0)),
            pl.BlockSpec((1, EMBED_DIM), lambda i: (0, 0)),
            pl.BlockSpec(memory_space=pltpu.SMEM),
        ],
        out_specs=pl.BlockSpec((1, 1, TBLOCK), lambda i: (i, 0, 0)),
        out_shape=jax.ShapeDtypeStruct(
            (NUM_EMB // TBLOCK, 1, TBLOCK), jnp.float32
        ),
    )(emb, w_scaled, b_scaled)


def _gather_sum(x2, p):
    mesh = plsc.VectorSubcoreMesh(core_axis_name="c", subcore_axis_name="s")

    @functools.partial(
        pl.kernel,
        out_type=jax.ShapeDtypeStruct((BATCH,), jnp.float32),
        mesh=mesh,
        compiler_params=pltpu.CompilerParams(needs_layout_passes=False),
        scratch_types=[
            pltpu.VMEM((IDX_ROWS, 128), jnp.int32),
            pltpu.VMEM((IDX_PER_CHUNK,), jnp.float32),
            pltpu.VMEM((ROWS_PER_W,), jnp.float32),
            pltpu.SemaphoreType.DMA,
        ],
    )
    def body(x2_hbm, p_hbm, y_hbm, idx_v, vals_v, out_v, sem):
        wid = lax.axis_index("c") * 16 + lax.axis_index("s")
        iota200 = lax.iota(jnp.int32, 16) * HIST

        def chunk_body(c, carry):
            xrow = wid * XROWS_PER_W + c * IDX_ROWS
            pltpu.sync_copy(x2_hbm.at[pl.ds(xrow, IDX_ROWS)], idx_v)

            def fire(j, carry2):
                for k in range(8):
                    r = j * 8 + k
                    pltpu.async_copy(
                        p_hbm.at[idx_v.at[r]],
                        vals_v.at[pl.ds(pl.multiple_of(r * 128, 128), 128)],
                        sem,
                    )
                return carry2

            lax.fori_loop(0, IDX_ROWS // 8, fire, 0)
            pltpu.make_async_copy(
                p_hbm.at[pl.ds(0, IDX_PER_CHUNK)], vals_v, sem
            ).wait()

            for g in range(8):
                base = g * 16 * HIST

                def red(j, acc):
                    jb = base + j * 8
                    vs = [
                        plsc.load_gather(vals_v, [iota200 + (jb + k)])
                        for k in range(8)
                    ]
                    s = ((vs[0] + vs[1]) + (vs[2] + vs[3])) + (
                        (vs[4] + vs[5]) + (vs[6] + vs[7])
                    )
                    return acc + s

                acc = lax.fori_loop(
                    0, HIST // 8, red, jnp.zeros((16,), jnp.float32)
                )
                out_v[pl.ds(pl.multiple_of(c * CHUNK_ROWS + g * 16, 16), 16)] = acc
            return carry

        lax.fori_loop(0, N_CHUNKS, chunk_body, 0)
        pltpu.sync_copy(
            out_v, y_hbm.at[pl.ds(pl.multiple_of(wid * ROWS_PER_W, 512), ROWS_PER_W)]
        )

    return body(x2, p)


def kernel(X, emb, W, b):
    x2 = X.astype(jnp.int32).reshape(BATCH * HIST // 128, 128)
    w_scaled = W.astype(jnp.float32).reshape(1, EMBED_DIM) * (1.0 / HIST)
    b_scaled = b.astype(jnp.float32).reshape(1) * (1.0 / HIST)
    p = _make_table(emb, w_scaled, b_scaled).reshape(NUM_EMB)
    y = _gather_sum(x2, p)
    return y.reshape(BATCH, 1)


# speedup vs baseline: 17.7281x; 1.1156x over previous
import functools

import jax
import jax.numpy as jnp
from jax import lax
from jax.experimental import pallas as pl
from jax.experimental.pallas import tpu as pltpu
from jax.experimental.pallas import tpu_sc as plsc

NUM_EMB = 1_000_000
EMBED_DIM = 32
BATCH = 16384
HIST = 200

NW = 32
ROWS_PER_W = BATCH // NW
CHUNK_ROWS = 128
N_CHUNKS = ROWS_PER_W // CHUNK_ROWS
IDX_PER_CHUNK = CHUNK_ROWS * HIST
IDX_ROWS = IDX_PER_CHUNK // 128
XROWS_PER_W = (BATCH * HIST) // 128 // NW

TBLOCK = 20000


def _table_body(emb_ref, w_ref, b_ref, out_ref):
    out_ref[0] = (
        lax.dot_general(
            w_ref[...],
            emb_ref[...],
            dimension_numbers=(((1,), (1,)), ((), ())),
            preferred_element_type=jnp.float32,
        )
        + b_ref[0]
    )


def _make_table(emb, w_scaled, b_scaled):
    grid = (NUM_EMB // TBLOCK,)
    return pl.pallas_call(
        _table_body,
        grid=grid,
        in_specs=[
            pl.BlockSpec((TBLOCK, EMBED_DIM), lambda i: (i, 0)),
            pl.BlockSpec((1, EMBED_DIM), lambda i: (0, 0)),
            pl.BlockSpec(memory_space=pltpu.SMEM),
        ],
        out_specs=pl.BlockSpec((1, 1, TBLOCK), lambda i: (i, 0, 0)),
        out_shape=jax.ShapeDtypeStruct(
            (NUM_EMB // TBLOCK, 1, TBLOCK), jnp.float32
        ),
    )(emb, w_scaled, b_scaled)


def _gather_sum(x2, p):
    mesh = plsc.VectorSubcoreMesh(core_axis_name="c", subcore_axis_name="s")

    @functools.partial(
        pl.kernel,
        out_type=jax.ShapeDtypeStruct((BATCH,), jnp.float32),
        mesh=mesh,
        compiler_params=pltpu.CompilerParams(needs_layout_passes=False),
        scratch_types=[
            pltpu.VMEM((IDX_ROWS, 128), jnp.int32),
            pltpu.VMEM((IDX_PER_CHUNK,), jnp.float32),
            pltpu.VMEM((ROWS_PER_W,), jnp.float32),
            pltpu.SemaphoreType.DMA,
        ],
    )
    def body(x2_hbm, p_hbm, y_hbm, idx_v, vals_v, out_v, sem):
        wid = lax.axis_index("c") * 16 + lax.axis_index("s")
        iota200 = lax.iota(jnp.int32, 16) * HIST

        def chunk_body(c, carry):
            xrow = wid * XROWS_PER_W + c * IDX_ROWS
            pltpu.sync_copy(x2_hbm.at[pl.ds(xrow, IDX_ROWS)], idx_v)

            def fire(j, carry2):
                for k in range(8):
                    r = j * 8 + k
                    pltpu.async_copy(
                        p_hbm.at[idx_v.at[r]],
                        vals_v.at[pl.ds(pl.multiple_of(r * 128, 128), 128)],
                        sem,
                    )
                return carry2

            lax.fori_loop(0, IDX_ROWS // 8, fire, 0)
            pltpu.make_async_copy(
                p_hbm.at[pl.ds(0, IDX_PER_CHUNK)], vals_v, sem
            ).wait()

            for g in range(8):
                base = g * 16 * HIST

                def red(j, acc):
                    jb = base + j * 8
                    vs = [
                        plsc.load_gather(vals_v, [iota200 + (jb + k)])
                        for k in range(8)
                    ]
                    s = ((vs[0] + vs[1]) + (vs[2] + vs[3])) + (
                        (vs[4] + vs[5]) + (vs[6] + vs[7])
                    )
                    return acc + s

                acc = lax.fori_loop(
                    0, HIST // 8, red, jnp.zeros((16,), jnp.float32)
                )
                out_v[pl.ds(pl.multiple_of(c * CHUNK_ROWS + g * 16, 16), 16)] = acc
            return carry

        lax.fori_loop(0, N_CHUNKS, chunk_body, 0)
        pltpu.sync_copy(
            out_v, y_hbm.at[pl.ds(pl.multiple_of(wid * ROWS_PER_W, 512), ROWS_PER_W)]
        )

    return body(x2, p)


def kernel(X, emb, W, b):
    x2 = X.astype(jnp.int32).reshape(BATCH * HIST // 128, 128)
    w_scaled = W.astype(jnp.float32).reshape(1, EMBED_DIM) * (1.0 / HIST)
    b_scaled = b.astype(jnp.float32).reshape(1) * (1.0 / HIST)
    p = _make_table(emb, w_scaled, b_scaled).reshape(NUM_EMB)
    y = _gather_sum(x2, p)
    return y.reshape(BATCH, 1)


# speedup vs baseline: 17.9922x; 1.0149x over previous
import functools

import jax
import jax.numpy as jnp
from jax import lax
from jax.experimental import pallas as pl
from jax.experimental.pallas import tpu as pltpu
from jax.experimental.pallas import tpu_sc as plsc

NUM_EMB = 1_000_000
EMBED_DIM = 32
BATCH = 16384
HIST = 200

NW = 32
ROWS_PER_W = BATCH // NW
CHUNK_ROWS = 128
N_CHUNKS = ROWS_PER_W // CHUNK_ROWS
IDX_PER_CHUNK = CHUNK_ROWS * HIST
IDX_ROWS = IDX_PER_CHUNK // 128
XROWS_PER_W = (BATCH * HIST) // 128 // NW

TBLOCK = 20000


def _table_body(emb_a_ref, emb_b_ref, w_ref, b_ref, out_ref):
    for h, ref in enumerate((emb_a_ref, emb_b_ref)):
        out_ref[0, 0, pl.ds(h * TBLOCK, TBLOCK)] = (
            lax.dot_general(
                w_ref[...],
                ref[...],
                dimension_numbers=(((1,), (1,)), ((), ())),
                preferred_element_type=jnp.float32,
            )[0]
            + b_ref[0]
        )


def _make_table(emb, w_scaled, b_scaled):
    grid = (NUM_EMB // (2 * TBLOCK),)
    return pl.pallas_call(
        _table_body,
        grid=grid,
        in_specs=[
            pl.BlockSpec((TBLOCK, EMBED_DIM), lambda i: (2 * i, 0)),
            pl.BlockSpec((TBLOCK, EMBED_DIM), lambda i: (2 * i + 1, 0)),
            pl.BlockSpec((1, EMBED_DIM), lambda i: (0, 0)),
            pl.BlockSpec(memory_space=pltpu.SMEM),
        ],
        out_specs=pl.BlockSpec((1, 1, 2 * TBLOCK), lambda i: (i, 0, 0)),
        out_shape=jax.ShapeDtypeStruct(
            (NUM_EMB // (2 * TBLOCK), 1, 2 * TBLOCK), jnp.float32
        ),
    )(emb, emb, w_scaled, b_scaled)


def _gather_sum(x2, p):
    mesh = plsc.VectorSubcoreMesh(core_axis_name="c", subcore_axis_name="s")

    @functools.partial(
        pl.kernel,
        out_type=jax.ShapeDtypeStruct((BATCH,), jnp.float32),
        mesh=mesh,
        compiler_params=pltpu.CompilerParams(needs_layout_passes=False),
        scratch_types=[
            pltpu.VMEM((IDX_ROWS, 128), jnp.int32),
            pltpu.VMEM((IDX_PER_CHUNK,), jnp.float32),
            pltpu.VMEM((ROWS_PER_W,), jnp.float32),
            pltpu.SemaphoreType.DMA,
        ],
    )
    def body(x2_hbm, p_hbm, y_hbm, idx_v, vals_v, out_v, sem):
        wid = lax.axis_index("c") * 16 + lax.axis_index("s")
        iota200 = lax.iota(jnp.int32, 16) * HIST

        def chunk_body(c, carry):
            xrow = wid * XROWS_PER_W + c * IDX_ROWS
            pltpu.sync_copy(x2_hbm.at[pl.ds(xrow, IDX_ROWS)], idx_v)

            def fire(j, carry2):
                for k in range(8):
                    r = j * 8 + k
                    pltpu.async_copy(
                        p_hbm.at[idx_v.at[r]],
                        vals_v.at[pl.ds(pl.multiple_of(r * 128, 128), 128)],
                        sem,
                    )
                return carry2

            lax.fori_loop(0, IDX_ROWS // 8, fire, 0)
            pltpu.make_async_copy(
                p_hbm.at[pl.ds(0, IDX_PER_CHUNK)], vals_v, sem
            ).wait()

            for g in range(8):
                base = g * 16 * HIST

                def red(j, acc):
                    jb = base + j * 8
                    vs = [
                        plsc.load_gather(vals_v, [iota200 + (jb + k)])
                        for k in range(8)
                    ]
                    s = ((vs[0] + vs[1]) + (vs[2] + vs[3])) + (
                        (vs[4] + vs[5]) + (vs[6] + vs[7])
                    )
                    return acc + s

                acc = lax.fori_loop(
                    0, HIST // 8, red, jnp.zeros((16,), jnp.float32)
                )
                out_v[pl.ds(pl.multiple_of(c * CHUNK_ROWS + g * 16, 16), 16)] = acc
            return carry

        lax.fori_loop(0, N_CHUNKS, chunk_body, 0)
        pltpu.sync_copy(
            out_v, y_hbm.at[pl.ds(pl.multiple_of(wid * ROWS_PER_W, 512), ROWS_PER_W)]
        )

    return body(x2, p)


def kernel(X, emb, W, b):
    x2 = X.astype(jnp.int32).reshape(BATCH * HIST // 128, 128)
    w_scaled = W.astype(jnp.float32).reshape(1, EMBED_DIM) * (1.0 / HIST)
    b_scaled = b.astype(jnp.float32).reshape(1) * (1.0 / HIST)
    p = _make_table(emb, w_scaled, b_scaled).reshape(NUM_EMB)
    y = _gather_sum(x2, p)
    return y.reshape(BATCH, 1)
